# baseline (speedup 1.0000x reference)
import functools

import jax
import jax.numpy as jnp
from jax import lax
from jax.experimental import pallas as pl
from jax.experimental.pallas import tpu as pltpu
from jax.experimental.pallas import tpu_sc as plsc

_N_UNIT = 100
_MAX_UNIT_SIZE = 800
_TABLE = _N_UNIT * _MAX_UNIT_SIZE
_B = 16384
_NC = 2
_NS = 16
_NW = _NC * _NS
_W = _B // _NW
_L = 16
_RB = 4
_NB = _N_UNIT // _RB
_RT = _N_UNIT - _NB * _RB
_PIECE = _RB * _MAX_UNIT_SIZE
_G = _W // _L


def _body(nums_hbm, table_hbm, out_hbm, table_v, xb, ob, shared,
          sin0, sin1, sout0, sout1, fan0, fan1, fan2, stin, stout, stfan):
    cid = lax.axis_index("c")
    sid = lax.axis_index("s")
    wid = sid * _NC + cid
    base = wid * _W

    def in_slice(r0, nr):
        return nums_hbm.at[pl.ds(r0, nr), pl.ds(base, _W)]

    def out_slice(r0, nr):
        return out_hbm.at[pl.ds(r0, nr), pl.ds(base, _W)]

    def piece(b):
        return pl.ds(b * _PIECE, _PIECE)

    fans = (fan0, fan1, fan2)

    pltpu.async_copy(in_slice(0, _RB), xb.at[0, :, :], sin0)
    pltpu.async_copy(in_slice(_RB, _RB), xb.at[1, :, :], sin1)
    if _RT:
        pltpu.async_copy(
            in_slice(_NB * _RB, _RT), xb.at[2, pl.ds(0, _RT), :], stin
        )

    @pl.when(sid == 0)
    def _():
        pltpu.sync_copy(table_hbm, shared)

    plsc.subcore_barrier()

    for f in range(3):
        pltpu.async_copy(shared.at[piece(f)], table_v.at[piece(f)], fans[f])
    if _RT:
        tail_piece = pl.ds(_NB * _PIECE, _RT * _MAX_UNIT_SIZE)
        pltpu.async_copy(shared.at[tail_piece], table_v.at[tail_piece], stfan)

    iota = lax.broadcasted_iota(jnp.int32, (_L,), 0)
    zeros = jnp.zeros((_L,), jnp.float32)

    def zero_mask_slots(r0, nr):
        unit = iota + r0
        plsc.store_scatter(
            table_v, [unit * _MAX_UNIT_SIZE], zeros, mask=iota < nr
        )

    def compute(s, b, nr):
        @plsc.parallel_loop(0, nr, 1, unroll=2)
        def _(r):
            shift = (b * _RB + r) * _MAX_UNIT_SIZE
            for k in range(_G):
                off = pl.ds(k * _L, _L)
                idx = xb[s, r, off] + shift
                v = plsc.load_gather(table_v, [idx])
                ob[s, r, off] = jnp.maximum(v, 0.0)

    def block(b, carry):
        s = lax.rem(b, 2)
        r0 = b * _RB
        xs = xb.at[s, :, :]
        os_ = ob.at[s, :, :]

        f = lax.rem(b, 3)
        for ff in range(3):
            @pl.when(f == ff)
            def _():
                pltpu.make_async_copy(
                    shared.at[piece(b)], table_v.at[piece(b)], fans[ff]
                ).wait()

        zero_mask_slots(r0, _RB)

        @pl.when(b <= _NB - 4)
        def _():
            bn = b + 3
            fn = lax.rem(bn, 3)
            for ff in range(3):
                @pl.when(fn == ff)
                def _():
                    pltpu.async_copy(
                        shared.at[piece(bn)], table_v.at[piece(bn)], fans[ff]
                    )

        @pl.when(s == 0)
        def _():
            pltpu.make_async_copy(in_slice(r0, _RB), xs, sin0).wait()

        @pl.when(s == 1)
        def _():
            pltpu.make_async_copy(in_slice(r0, _RB), xs, sin1).wait()

        @pl.when(b >= 2)
        def _():
            @pl.when(s == 0)
            def _():
                pltpu.make_async_copy(
                    os_, out_slice(r0 - 2 * _RB, _RB), sout0
                ).wait()

            @pl.when(s == 1)
            def _():
                pltpu.make_async_copy(
                    os_, out_slice(r0 - 2 * _RB, _RB), sout1
                ).wait()

        compute(s, b, _RB)

        @pl.when(s == 0)
        def _():
            pltpu.async_copy(os_, out_slice(r0, _RB), sout0)

        @pl.when(s == 1)
        def _():
            pltpu.async_copy(os_, out_slice(r0, _RB), sout1)

        @pl.when(b + 2 < _NB)
        def _():
            @pl.when(s == 0)
            def _():
                pltpu.async_copy(in_slice(r0 + 2 * _RB, _RB), xs, sin0)

            @pl.when(s == 1)
            def _():
                pltpu.async_copy(in_slice(r0 + 2 * _RB, _RB), xs, sin1)

        return carry

    lax.fori_loop(0, _NB, block, 0)

    if _RT:
        r0 = _NB * _RB
        pltpu.make_async_copy(
            shared.at[tail_piece], table_v.at[tail_piece], stfan
        ).wait()
        zero_mask_slots(r0, _RT)
        pltpu.make_async_copy(
            in_slice(r0, _RT), xb.at[2, pl.ds(0, _RT), :], stin
        ).wait()

        @plsc.parallel_loop(0, _RT, 1, unroll=1)
        def _(r):
            shift = (r0 + r) * _MAX_UNIT_SIZE
            for k in range(_G):
                off = pl.ds(k * _L, _L)
                idx = xb[2, r, off] + shift
                v = plsc.load_gather(table_v, [idx])
                ob[2, r, off] = jnp.maximum(v, 0.0)

        pltpu.async_copy(ob.at[2, pl.ds(0, _RT), :], out_slice(r0, _RT), stout)

    pltpu.make_async_copy(
        ob.at[0, :, :], out_slice((_NB - 2) * _RB, _RB), sout0
    ).wait()
    pltpu.make_async_copy(
        ob.at[1, :, :], out_slice((_NB - 1) * _RB, _RB), sout1
    ).wait()
    if _RT:
        pltpu.make_async_copy(
            ob.at[2, pl.ds(0, _RT), :], out_slice(_NB * _RB, _RT), stout
        ).wait()


@jax.jit
def _run(nums_t, table_flat):
    mesh = plsc.VectorSubcoreMesh(core_axis_name="c", subcore_axis_name="s")
    return pl.kernel(
        _body,
        mesh=mesh,
        compiler_params=pltpu.CompilerParams(needs_layout_passes=False),
        out_type=jax.ShapeDtypeStruct((_N_UNIT, _B), jnp.float32),
        scratch_types=[
            pltpu.VMEM((_TABLE,), jnp.float32),
            pltpu.VMEM((3, _RB, _W), jnp.int32),
            pltpu.VMEM((3, _RB, _W), jnp.float32),
            pltpu.VMEM_SHARED((_TABLE,), jnp.float32),
            pltpu.SemaphoreType.DMA,
            pltpu.SemaphoreType.DMA,
            pltpu.SemaphoreType.DMA,
            pltpu.SemaphoreType.DMA,
            pltpu.SemaphoreType.DMA,
            pltpu.SemaphoreType.DMA,
            pltpu.SemaphoreType.DMA,
            pltpu.SemaphoreType.DMA,
            pltpu.SemaphoreType.DMA,
            pltpu.SemaphoreType.DMA,
        ],
    )(nums_t, table_flat)


def kernel(unit_nums, embed_weight):
    out_t = _run(unit_nums.T, embed_weight[:, 0])
    return out_t.T


# speedup vs baseline: 1.0578x; 1.0578x over previous
import functools

import jax
import jax.numpy as jnp
from jax import lax
from jax.experimental import pallas as pl
from jax.experimental.pallas import tpu as pltpu
from jax.experimental.pallas import tpu_sc as plsc

_N_UNIT = 100
_MAX_UNIT_SIZE = 800
_TABLE = _N_UNIT * _MAX_UNIT_SIZE
_B = 16384
_NC = 2
_NS = 16
_NW = _NC * _NS
_W = _B // _NW
_L = 16
_RB = 24
_NRB = 4
_CH = 128
_NCC = _W // _CH
_NCHUNK = _NRB * _NCC
_RT = _N_UNIT - _NRB * _RB
_PIECE = _RB * _MAX_UNIT_SIZE
_G = _CH // _L


def _body(nums_hbm, table_hbm, out_hbm, table_v, xb, ob, xt, ot, shared,
          sin0, sin1, sout0, sout1, fan0, fan1, fan2, stin, stout, stfan):
    cid = lax.axis_index("c")
    sid = lax.axis_index("s")
    wid = sid * _NC + cid
    base = wid * _W

    fans = (fan0, fan1, fan2)

    def io_slice(ref, cc):
        rb = lax.div(cc, _NCC)
        col = lax.rem(cc, _NCC)
        return ref.at[
            pl.ds(rb * _RB, _RB), pl.ds(base + col * _CH, _CH)
        ]

    def piece(rb):
        return pl.ds(rb * _PIECE, _PIECE)

    pltpu.async_copy(io_slice(nums_hbm, 0), xb.at[0], sin0)
    pltpu.async_copy(io_slice(nums_hbm, 1), xb.at[1], sin1)
    tail_rows = pl.ds(_NRB * _RB, _RT)
    pltpu.async_copy(nums_hbm.at[tail_rows, pl.ds(base, _W)], xt, stin)

    @pl.when(sid == 0)
    def _():
        pltpu.sync_copy(table_hbm, shared)

    plsc.subcore_barrier()

    for f in range(2):
        pltpu.async_copy(shared.at[piece(f)], table_v.at[piece(f)], fans[f])
    tail_piece = pl.ds(_NRB * _PIECE, _RT * _MAX_UNIT_SIZE)
    pltpu.async_copy(shared.at[tail_piece], table_v.at[tail_piece], stfan)

    iota = lax.broadcasted_iota(jnp.int32, (_L,), 0)
    zeros = jnp.zeros((_L,), jnp.float32)

    def zero_mask_slots(r0, nr):
        for g in range((_RB + _L - 1) // _L):
            unit = iota + (r0 + g * _L)
            plsc.store_scatter(
                table_v, [unit * _MAX_UNIT_SIZE], zeros,
                mask=iota + g * _L < nr,
            )

    def chunk(cc, carry):
        s = lax.rem(cc, 2)
        rb = lax.div(cc, _NCC)
        col = lax.rem(cc, _NCC)
        r0 = rb * _RB

        @pl.when(col == 0)
        def _():
            f = lax.rem(rb, 3)
            for ff in range(3):
                @pl.when(f == ff)
                def _():
                    pltpu.make_async_copy(
                        shared.at[piece(rb)], table_v.at[piece(rb)], fans[ff]
                    ).wait()

            zero_mask_slots(r0, _RB)

            @pl.when(rb < _NRB - 2)
            def _():
                rn = rb + 2
                fn = lax.rem(rn, 3)
                for ff in range(3):
                    @pl.when(fn == ff)
                    def _():
                        pltpu.async_copy(
                            shared.at[piece(rn)], table_v.at[piece(rn)],
                            fans[ff],
                        )

        @pl.when(s == 0)
        def _():
            pltpu.make_async_copy(io_slice(nums_hbm, cc), xb.at[0], sin0).wait()

        @pl.when(s == 1)
        def _():
            pltpu.make_async_copy(io_slice(nums_hbm, cc), xb.at[1], sin1).wait()

        @pl.when(cc >= 2)
        def _():
            @pl.when(s == 0)
            def _():
                pltpu.make_async_copy(
                    ob.at[0], io_slice(out_hbm, cc - 2), sout0
                ).wait()

            @pl.when(s == 1)
            def _():
                pltpu.make_async_copy(
                    ob.at[1], io_slice(out_hbm, cc - 2), sout1
                ).wait()

        @plsc.parallel_loop(0, _RB, 1, unroll=2)
        def _(r):
            shift = (r0 + r) * _MAX_UNIT_SIZE
            for k in range(_G):
                off = pl.ds(k * _L, _L)
                idx = xb[s, r, off] + shift
                v = plsc.load_gather(table_v, [idx])
                ob[s, r, off] = jnp.maximum(v, 0.0)

        @pl.when(s == 0)
        def _():
            pltpu.async_copy(ob.at[0], io_slice(out_hbm, cc), sout0)

        @pl.when(s == 1)
        def _():
            pltpu.async_copy(ob.at[1], io_slice(out_hbm, cc), sout1)

        @pl.when(cc + 2 < _NCHUNK)
        def _():
            @pl.when(s == 0)
            def _():
                pltpu.async_copy(io_slice(nums_hbm, cc + 2), xb.at[0], sin0)

            @pl.when(s == 1)
            def _():
                pltpu.async_copy(io_slice(nums_hbm, cc + 2), xb.at[1], sin1)

        return carry

    lax.fori_loop(0, _NCHUNK, chunk, 0)

    pltpu.make_async_copy(
        shared.at[tail_piece], table_v.at[tail_piece], stfan
    ).wait()
    zero_mask_slots(_NRB * _RB, _RT)
    pltpu.make_async_copy(
        nums_hbm.at[tail_rows, pl.ds(base, _W)], xt, stin
    ).wait()

    @plsc.parallel_loop(0, _RT, 1, unroll=2)
    def _(r):
        shift = (_NRB * _RB + r) * _MAX_UNIT_SIZE
        for k in range(_W // _L):
            off = pl.ds(k * _L, _L)
            idx = xt[r, off] + shift
            v = plsc.load_gather(table_v, [idx])
            ot[r, off] = jnp.maximum(v, 0.0)

    pltpu.async_copy(ot, out_hbm.at[tail_rows, pl.ds(base, _W)], stout)

    pltpu.make_async_copy(
        ob.at[0], io_slice(out_hbm, _NCHUNK - 2), sout0
    ).wait()
    pltpu.make_async_copy(
        ob.at[1], io_slice(out_hbm, _NCHUNK - 1), sout1
    ).wait()
    pltpu.make_async_copy(
        ot, out_hbm.at[tail_rows, pl.ds(base, _W)], stout
    ).wait()


@jax.jit
def _run(nums_t, table_flat):
    mesh = plsc.VectorSubcoreMesh(core_axis_name="c", subcore_axis_name="s")
    return pl.kernel(
        _body,
        mesh=mesh,
        compiler_params=pltpu.CompilerParams(needs_layout_passes=False),
        out_type=jax.ShapeDtypeStruct((_N_UNIT, _B), jnp.float32),
        scratch_types=[
            pltpu.VMEM((_TABLE,), jnp.float32),
            pltpu.VMEM((2, _RB, _CH), jnp.int32),
            pltpu.VMEM((2, _RB, _CH), jnp.float32),
            pltpu.VMEM((_RT, _W), jnp.int32),
            pltpu.VMEM((_RT, _W), jnp.float32),
            pltpu.VMEM_SHARED((_TABLE,), jnp.float32),
            pltpu.SemaphoreType.DMA,
            pltpu.SemaphoreType.DMA,
            pltpu.SemaphoreType.DMA,
            pltpu.SemaphoreType.DMA,
            pltpu.SemaphoreType.DMA,
            pltpu.SemaphoreType.DMA,
            pltpu.SemaphoreType.DMA,
            pltpu.SemaphoreType.DMA,
            pltpu.SemaphoreType.DMA,
            pltpu.SemaphoreType.DMA,
        ],
    )(nums_t, table_flat)


def kernel(unit_nums, embed_weight):
    out_t = _run(unit_nums.T, embed_weight[:, 0])
    return out_t.T


# speedup vs baseline: 1.1175x; 1.0564x over previous
import functools

import jax
import jax.numpy as jnp
from jax import lax
from jax.experimental import pallas as pl
from jax.experimental.pallas import tpu as pltpu
from jax.experimental.pallas import tpu_sc as plsc

_N_UNIT = 100
_MAX_UNIT_SIZE = 800
_TABLE = _N_UNIT * _MAX_UNIT_SIZE
_B = 16384
_NC = 2
_NS = 16
_NW = _NC * _NS
_COLS_W = _B // _NW
_L = 16
_CH = 128
_NCH = _COLS_W // _CH
_R1 = 48
_R2 = _N_UNIT - _R1
_RB = 56
_SPLIT = _R1 * _MAX_UNIT_SIZE


def _body(nums_hbm, table_hbm, out_hbm, tv1, tv2, xb, ob, shared,
          sin0, sin1, sout0, sout1, semt):
    cid = lax.axis_index("c")
    sid = lax.axis_index("s")
    wid = sid * _NC + cid
    base = wid * _COLS_W

    sins = (sin0, sin1)
    souts = (sout0, sout1)
    rows = (_R1, _R2)

    def in_slice(phase, cc, nr):
        return nums_hbm.at[
            pl.ds(phase * _R1, nr), pl.ds(base + cc * _CH, _CH)
        ]

    def out_slice(phase, cc, nr):
        return out_hbm.at[
            pl.ds(phase * _R1, nr), pl.ds(base + cc * _CH, _CH)
        ]

    pltpu.async_copy(in_slice(0, 0, _R1), xb.at[0, pl.ds(0, _R1), :], sin0)
    pltpu.async_copy(in_slice(0, 1, _R1), xb.at[1, pl.ds(0, _R1), :], sin1)

    @pl.when(sid == 0)
    def _():
        pltpu.sync_copy(table_hbm, shared)

    plsc.subcore_barrier()
    pltpu.sync_copy(shared.at[pl.ds(0, _SPLIT)], tv1)
    h2 = pltpu.async_copy(shared.at[pl.ds(_SPLIT, _TABLE - _SPLIT)], tv2, semt)

    iota = lax.broadcasted_iota(jnp.int32, (_L,), 0)
    zeros = jnp.zeros((_L,), jnp.float32)

    def zero_mask_slots(tv, nr):
        for g in range((nr + _L - 1) // _L):
            unit = iota + g * _L
            plsc.store_scatter(
                tv, [unit * _MAX_UNIT_SIZE], zeros, mask=unit < nr
            )

    zero_mask_slots(tv1, _R1)

    def run_phase(phase, tv, nr):
        def chunk(cc, carry):
            s = lax.rem(cc, 2)
            xs = xb.at[s, pl.ds(0, nr), :]
            os_ = ob.at[s, pl.ds(0, nr), :]

            @pl.when(s == 0)
            def _():
                pltpu.make_async_copy(in_slice(phase, cc, nr), xs, sin0).wait()

            @pl.when(s == 1)
            def _():
                pltpu.make_async_copy(in_slice(phase, cc, nr), xs, sin1).wait()

            @pl.when(cc >= 2)
            def _():
                @pl.when(s == 0)
                def _():
                    pltpu.make_async_copy(
                        os_, out_slice(phase, cc - 2, nr), sout0
                    ).wait()

                @pl.when(s == 1)
                def _():
                    pltpu.make_async_copy(
                        os_, out_slice(phase, cc - 2, nr), sout1
                    ).wait()

            @plsc.parallel_loop(0, nr, 1, unroll=4)
            def _(r):
                shift = r * _MAX_UNIT_SIZE
                for k in range(_CH // _L):
                    off = pl.ds(k * _L, _L)
                    idx = xb[s, r, off] + shift
                    v = plsc.load_gather(tv, [idx])
                    ob[s, r, off] = jnp.maximum(v, 0.0)

            @pl.when(s == 0)
            def _():
                pltpu.async_copy(os_, out_slice(phase, cc, nr), sout0)

            @pl.when(s == 1)
            def _():
                pltpu.async_copy(os_, out_slice(phase, cc, nr), sout1)

            @pl.when(cc + 2 < _NCH)
            def _():
                @pl.when(s == 0)
                def _():
                    pltpu.async_copy(in_slice(phase, cc + 2, nr), xs, sin0)

                @pl.when(s == 1)
                def _():
                    pltpu.async_copy(in_slice(phase, cc + 2, nr), xs, sin1)

            return carry

        lax.fori_loop(0, _NCH, chunk, 0)
        for cc in (_NCH - 2, _NCH - 1):
            s = cc % 2
            pltpu.make_async_copy(
                ob.at[s, pl.ds(0, nr), :],
                out_slice(phase, cc, nr),
                souts[s],
            ).wait()

    run_phase(0, tv1, _R1)

    h2.wait()
    zero_mask_slots(tv2, _R2)
    pltpu.async_copy(in_slice(1, 0, _R2), xb.at[0, pl.ds(0, _R2), :], sin0)
    pltpu.async_copy(in_slice(1, 1, _R2), xb.at[1, pl.ds(0, _R2), :], sin1)
    run_phase(1, tv2, _R2)


@jax.jit
def _run(nums_t, table_flat):
    mesh = plsc.VectorSubcoreMesh(core_axis_name="c", subcore_axis_name="s")
    return pl.kernel(
        _body,
        mesh=mesh,
        compiler_params=pltpu.CompilerParams(needs_layout_passes=False),
        out_type=jax.ShapeDtypeStruct((_N_UNIT, _B), jnp.float32),
        scratch_types=[
            pltpu.VMEM((_SPLIT,), jnp.float32),
            pltpu.VMEM((_TABLE - _SPLIT,), jnp.float32),
            pltpu.VMEM((2, _RB, _CH), jnp.int32),
            pltpu.VMEM((2, _RB, _CH), jnp.float32),
            pltpu.VMEM_SHARED((_TABLE,), jnp.float32),
            pltpu.SemaphoreType.DMA,
            pltpu.SemaphoreType.DMA,
            pltpu.SemaphoreType.DMA,
            pltpu.SemaphoreType.DMA,
            pltpu.SemaphoreType.DMA,
        ],
    )(nums_t, table_flat)


def kernel(unit_nums, embed_weight):
    out_t = _run(unit_nums.T, embed_weight[:, 0])
    return out_t.T


# speedup vs baseline: 1.1242x; 1.0060x over previous
import functools

import jax
import jax.numpy as jnp
from jax import lax
from jax.experimental import pallas as pl
from jax.experimental.pallas import tpu as pltpu
from jax.experimental.pallas import tpu_sc as plsc

_N_UNIT = 100
_MAX_UNIT_SIZE = 800
_TABLE = _N_UNIT * _MAX_UNIT_SIZE
_B = 16384
_NC = 2
_NS = 16
_NW = _NC * _NS
_COLS_W = _B // _NW
_L = 16
_CH = 128
_NCH = _COLS_W // _CH
_R1 = 48
_R2 = _N_UNIT - _R1
_RB = 56
_SPLIT = _R1 * _MAX_UNIT_SIZE


def _body(nums_hbm, table_hbm, out_hbm, tv1, tv2, xb, ob, shared,
          sin0, sin1, sout0, sout1, semt):
    cid = lax.axis_index("c")
    sid = lax.axis_index("s")
    wid = sid * _NC + cid
    base = wid * _COLS_W

    sins = (sin0, sin1)
    souts = (sout0, sout1)
    rows = (_R1, _R2)

    def in_slice(phase, cc, nr):
        return nums_hbm.at[
            pl.ds(phase * _R1, nr), pl.ds(base + cc * _CH, _CH)
        ]

    def out_slice(phase, cc, nr):
        return out_hbm.at[
            pl.ds(phase * _R1, nr), pl.ds(base + cc * _CH, _CH)
        ]

    pltpu.async_copy(in_slice(0, 0, _R1), xb.at[0, pl.ds(0, _R1), :], sin0)
    pltpu.async_copy(in_slice(0, 1, _R1), xb.at[1, pl.ds(0, _R1), :], sin1)

    @pl.when(sid == 0)
    def _():
        pltpu.sync_copy(table_hbm, shared)

    plsc.subcore_barrier()
    pltpu.sync_copy(shared.at[pl.ds(0, _SPLIT)], tv1)
    h2 = pltpu.async_copy(shared.at[pl.ds(_SPLIT, _TABLE - _SPLIT)], tv2, semt)

    iota = lax.broadcasted_iota(jnp.int32, (_L,), 0)
    zeros = jnp.zeros((_L,), jnp.float32)

    def zero_mask_slots(tv, nr):
        for g in range((nr + _L - 1) // _L):
            unit = iota + g * _L
            plsc.store_scatter(
                tv, [unit * _MAX_UNIT_SIZE], zeros, mask=unit < nr
            )

    zero_mask_slots(tv1, _R1)

    def run_phase(phase, tv, nr):
        def chunk(cc, carry):
            s = lax.rem(cc, 2)
            xs = xb.at[s, pl.ds(0, nr), :]
            os_ = ob.at[s, pl.ds(0, nr), :]

            @pl.when(s == 0)
            def _():
                pltpu.make_async_copy(in_slice(phase, cc, nr), xs, sin0).wait()

            @pl.when(s == 1)
            def _():
                pltpu.make_async_copy(in_slice(phase, cc, nr), xs, sin1).wait()

            @pl.when(cc >= 2)
            def _():
                @pl.when(s == 0)
                def _():
                    pltpu.make_async_copy(
                        os_, out_slice(phase, cc - 2, nr), sout0
                    ).wait()

                @pl.when(s == 1)
                def _():
                    pltpu.make_async_copy(
                        os_, out_slice(phase, cc - 2, nr), sout1
                    ).wait()

            @plsc.parallel_loop(0, nr, 1, unroll=2)
            def _(r):
                shift = r * _MAX_UNIT_SIZE
                for k in range(_CH // _L):
                    off = pl.ds(k * _L, _L)
                    idx = xb[s, r, off] + shift
                    v = plsc.load_gather(tv, [idx])
                    ob[s, r, off] = jnp.maximum(v, 0.0)

            @pl.when(s == 0)
            def _():
                pltpu.async_copy(os_, out_slice(phase, cc, nr), sout0)

            @pl.when(s == 1)
            def _():
                pltpu.async_copy(os_, out_slice(phase, cc, nr), sout1)

            @pl.when(cc + 2 < _NCH)
            def _():
                @pl.when(s == 0)
                def _():
                    pltpu.async_copy(in_slice(phase, cc + 2, nr), xs, sin0)

                @pl.when(s == 1)
                def _():
                    pltpu.async_copy(in_slice(phase, cc + 2, nr), xs, sin1)

            return carry

        lax.fori_loop(0, _NCH, chunk, 0)
        for cc in (_NCH - 2, _NCH - 1):
            s = cc % 2
            pltpu.make_async_copy(
                ob.at[s, pl.ds(0, nr), :],
                out_slice(phase, cc, nr),
                souts[s],
            ).wait()

    run_phase(0, tv1, _R1)

    h2.wait()
    zero_mask_slots(tv2, _R2)
    pltpu.async_copy(in_slice(1, 0, _R2), xb.at[0, pl.ds(0, _R2), :], sin0)
    pltpu.async_copy(in_slice(1, 1, _R2), xb.at[1, pl.ds(0, _R2), :], sin1)
    run_phase(1, tv2, _R2)


@jax.jit
def _run(nums_t, table_flat):
    mesh = plsc.VectorSubcoreMesh(core_axis_name="c", subcore_axis_name="s")
    return pl.kernel(
        _body,
        mesh=mesh,
        compiler_params=pltpu.CompilerParams(needs_layout_passes=False),
        out_type=jax.ShapeDtypeStruct((_N_UNIT, _B), jnp.float32),
        scratch_types=[
            pltpu.VMEM((_SPLIT,), jnp.float32),
            pltpu.VMEM((_TABLE - _SPLIT,), jnp.float32),
            pltpu.VMEM((2, _RB, _CH), jnp.int32),
            pltpu.VMEM((2, _RB, _CH), jnp.float32),
            pltpu.VMEM_SHARED((_TABLE,), jnp.float32),
            pltpu.SemaphoreType.DMA,
            pltpu.SemaphoreType.DMA,
            pltpu.SemaphoreType.DMA,
            pltpu.SemaphoreType.DMA,
            pltpu.SemaphoreType.DMA,
        ],
    )(nums_t, table_flat)


def kernel(unit_nums, embed_weight):
    out_t = _run(unit_nums.T, embed_weight[:, 0])
    return out_t.T
